# baseline (speedup 1.0000x reference)
import jax
import jax.numpy as jnp
from jax.experimental import pallas as pl
from jax.experimental.pallas import tpu as pltpu

B = 8
M = 2048
FDIM = 256
N = 8192
TILE = 512
NEG = -1000000000.0


def _body(ids_ref, x_ref, dense_ref, dmask_ref, attn_ref):
    b = pl.program_id(0)
    t = pl.program_id(1)
    ids = ids_ref[...]
    count = jnp.sum((ids == b).astype(jnp.int32))
    start = jnp.sum((ids < b).astype(jnp.int32))

    jw = t * TILE
    src = jnp.minimum(start + jw, N)
    src8 = pl.multiple_of(jnp.minimum((src // 8) * 8, N - TILE - 8), 8)
    shift = src - src8
    rows = x_ref[pl.ds(src8, TILE + 8), :]
    rows = pltpu.roll(rows, (TILE + 8) - shift, 0)[:TILE, :]
    j = jw + jax.lax.broadcasted_iota(jnp.int32, (TILE, 1), 0)
    dense_ref[0, :, :] = jnp.where(j < count, rows, 0.0)

    col = jax.lax.broadcasted_iota(jnp.int32, (1, M), 1)
    valid_row = col < count
    @pl.when(jnp.logical_and(b == 0, t == 0))
    def _write_dmask():
        counts = jnp.concatenate(
            [
                jnp.full((1, 1), jnp.sum((ids == k).astype(jnp.int32)), jnp.int32)
                for k in range(B)
            ],
            axis=0,
        )
        colb = jax.lax.broadcasted_iota(jnp.int32, (B, M), 1)
        dmask_ref[...] = colb < counts
    attn_ref[0, 0, :, :] = jnp.broadcast_to(
        jnp.where(valid_row, 0.0, NEG), (TILE, M)
    )


def kernel(x, batch_ids):
    ids2d = batch_ids.reshape(64, 128)
    dense, dmask, attn = pl.pallas_call(
        _body,
        grid=(B, M // TILE),
        in_specs=[
            pl.BlockSpec((64, 128), lambda b, t: (0, 0)),
            pl.BlockSpec((N, FDIM), lambda b, t: (0, 0)),
        ],
        out_specs=[
            pl.BlockSpec((1, TILE, FDIM), lambda b, t: (b, t, 0)),
            pl.BlockSpec((B, M), lambda b, t: (0, 0)),
            pl.BlockSpec((1, 1, TILE, M), lambda b, t: (b, 0, t, 0)),
        ],
        out_shape=[
            jax.ShapeDtypeStruct((B, M, FDIM), jnp.float32),
            jax.ShapeDtypeStruct((B, M), jnp.bool_),
            jax.ShapeDtypeStruct((B, 1, M, M), jnp.float32),
        ],
    )(ids2d, x)
    return dense, dmask, attn


# speedup vs baseline: 1.1217x; 1.1217x over previous
import jax
import jax.numpy as jnp
from jax.experimental import pallas as pl
from jax.experimental.pallas import tpu as pltpu

B = 8
M = 2048
FDIM = 256
N = 8192
TILE = 512
NEG = -1000000000.0


def _body(ids_ref, x_ref, dense_ref, dmask_ref, attn_ref):
    b = pl.program_id(0)
    t = pl.program_id(1)
    ids = ids_ref[...]
    count = jnp.sum((ids == b).astype(jnp.int32))
    start = jnp.sum((ids < b).astype(jnp.int32))

    jw = t * TILE
    src = jnp.minimum(start + jw, N)
    src8 = pl.multiple_of(jnp.minimum((src // 8) * 8, N - TILE - 8), 8)
    shift = src - src8
    rows = x_ref[pl.ds(src8, TILE + 8), :]
    rows = pltpu.roll(rows, (TILE + 8) - shift, 0)[:TILE, :]
    j = jw + jax.lax.broadcasted_iota(jnp.int32, (TILE, 1), 0)
    dense_ref[0, :, :] = jnp.where(j < count, rows, 0.0)

    col = jax.lax.broadcasted_iota(jnp.int32, (1, M), 1)
    valid_row = col < count
    dmask_ref[0, 0, :] = valid_row[0, :]
    attn_ref[0, 0, :, :] = jnp.broadcast_to(
        jnp.where(valid_row, 0.0, NEG), (TILE, M)
    )


def kernel(x, batch_ids):
    ids2d = batch_ids.reshape(64, 128)
    dense, dmask3, attn = pl.pallas_call(
        _body,
        grid=(B, M // TILE),
        in_specs=[
            pl.BlockSpec((64, 128), lambda b, t: (0, 0)),
            pl.BlockSpec((N, FDIM), lambda b, t: (0, 0)),
        ],
        out_specs=[
            pl.BlockSpec((1, TILE, FDIM), lambda b, t: (b, t, 0)),
            pl.BlockSpec((1, 1, M), lambda b, t: (b, 0, 0)),
            pl.BlockSpec((1, 1, TILE, M), lambda b, t: (b, 0, t, 0)),
        ],
        out_shape=[
            jax.ShapeDtypeStruct((B, M, FDIM), jnp.float32),
            jax.ShapeDtypeStruct((B, 1, M), jnp.bool_),
            jax.ShapeDtypeStruct((B, 1, M, M), jnp.float32),
        ],
    )(ids2d, x)
    return dense, dmask3.reshape(B, M), attn
